# Initial kernel scaffold; baseline (speedup 1.0000x reference)
import jax
import jax.numpy as jnp
from jax.experimental import pallas as pl


def kernel(coords_1, coords_2, features_1, features_2, W0, b0, g0, bt0, W1, b1, g1, bt1, W2, b2, g2, bt2):
    raise NotImplementedError("write your pallas kernel here")


# speedup vs baseline: 7.8062x; 7.8062x over previous
import functools

import jax
import jax.numpy as jnp
from jax import lax
from jax.experimental import pallas as pl

BN_BLK = 256


def _topk_interp_l0_kernel(c1t_ref, c2_ref, f1_ref, F2_ref, W0_ref, b0_ref,
                           z0_ref, stats_ref):
    bi = pl.program_id(0)
    ni = pl.program_id(1)
    c1 = c1t_ref[0]
    c2 = c2_ref[0]
    M = c2.shape[0]
    BNb = c1.shape[1]
    d = None
    for a in range(3):
        c1a = c1[a:a + 1, :]
        c2a = c2[:, a:a + 1]
        t = (c2a - c1a) ** 2
        d = t if d is None else d + t
    iota0 = lax.broadcasted_iota(jnp.int32, (M, BNb), 0)
    S = jnp.zeros((M, BNb), jnp.float32)
    norm = jnp.zeros((1, BNb), jnp.float32)
    for _ in range(3):
        m = jnp.min(d, axis=0, keepdims=True)
        elig = d == m
        idxk = jnp.min(jnp.where(elig, iota0, M), axis=0, keepdims=True)
        onehot = iota0 == idxk
        wk = 1.0 / (m + 1e-9)
        S = S + jnp.where(onehot, wk, 0.0)
        norm = norm + wk
        d = jnp.where(onehot, jnp.float32(jnp.inf), d)
    S = S / norm
    F2 = F2_ref[0]
    f2 = lax.dot_general(S, F2, (((0,), (0,)), ((), ())),
                         preferred_element_type=jnp.float32)
    f1 = f1_ref[0]
    W0 = W0_ref[...]
    C1 = f1.shape[1]
    z = lax.dot_general(f1, W0[:, :C1], (((1,), (1,)), ((), ())),
                        preferred_element_type=jnp.float32)
    z = z + lax.dot_general(f2, W0[:, C1:], (((1,), (1,)), ((), ())),
                            preferred_element_type=jnp.float32)
    z = z + b0_ref[...]
    z0_ref[0] = z

    @pl.when((bi == 0) & (ni == 0))
    def _():
        stats_ref[...] = jnp.zeros_like(stats_ref)

    stats_ref[...] += jnp.concatenate(
        [jnp.sum(z, axis=0, keepdims=True),
         jnp.sum(z * z, axis=0, keepdims=True)], axis=0)


def _bn_mlp_kernel(x_ref, stats_ref, g_ref, bt_ref, W_ref, b_ref,
                   z_ref, ostats_ref, *, count):
    bi = pl.program_id(0)
    ni = pl.program_id(1)
    mean = stats_ref[0:1, :] / count
    var = stats_ref[1:2, :] / count - mean * mean
    scale = g_ref[...] * lax.rsqrt(var + 1e-5)
    shift = bt_ref[...] - mean * scale
    y = jnp.maximum(x_ref[0] * scale + shift, 0.0)
    z = lax.dot_general(y, W_ref[...], (((1,), (1,)), ((), ())),
                        preferred_element_type=jnp.float32) + b_ref[...]
    z_ref[0] = z

    @pl.when((bi == 0) & (ni == 0))
    def _():
        ostats_ref[...] = jnp.zeros_like(ostats_ref)

    ostats_ref[...] += jnp.concatenate(
        [jnp.sum(z, axis=0, keepdims=True),
         jnp.sum(z * z, axis=0, keepdims=True)], axis=0)


def _bn_relu_kernel(x_ref, stats_ref, g_ref, bt_ref, o_ref, *, count):
    mean = stats_ref[0:1, :] / count
    var = stats_ref[1:2, :] / count - mean * mean
    scale = g_ref[...] * lax.rsqrt(var + 1e-5)
    shift = bt_ref[...] - mean * scale
    o_ref[0] = jnp.maximum(x_ref[0] * scale + shift, 0.0)


def _row2(r):
    return r.reshape(1, -1)


def kernel(coords_1, coords_2, features_1, features_2,
           W0, b0, g0, bt0, W1, b1, g1, bt1, W2, b2, g2, bt2):
    B, N, _ = coords_1.shape
    M = coords_2.shape[1]
    C1 = features_1.shape[2]
    C2 = features_2.shape[2]
    H0 = W0.shape[0]
    H1 = W1.shape[0]
    H2 = W2.shape[0]
    count = float(B * N)
    grid = (B, N // BN_BLK)
    f32 = jnp.float32

    c1t = jnp.transpose(coords_1, (0, 2, 1))

    z0, s0 = pl.pallas_call(
        _topk_interp_l0_kernel,
        grid=grid,
        in_specs=[
            pl.BlockSpec((1, 3, BN_BLK), lambda b, n: (b, 0, n)),
            pl.BlockSpec((1, M, 3), lambda b, n: (b, 0, 0)),
            pl.BlockSpec((1, BN_BLK, C1), lambda b, n: (b, n, 0)),
            pl.BlockSpec((1, M, C2), lambda b, n: (b, 0, 0)),
            pl.BlockSpec((H0, C1 + C2), lambda b, n: (0, 0)),
            pl.BlockSpec((1, H0), lambda b, n: (0, 0)),
        ],
        out_specs=[
            pl.BlockSpec((1, BN_BLK, H0), lambda b, n: (b, n, 0)),
            pl.BlockSpec((2, H0), lambda b, n: (0, 0)),
        ],
        out_shape=[
            jax.ShapeDtypeStruct((B, N, H0), f32),
            jax.ShapeDtypeStruct((2, H0), f32),
        ],
    )(c1t, coords_2, features_1, features_2, W0, _row2(b0))

    def mlp_layer(x, stats, g, bt, W, b, Cin, Cout):
        return pl.pallas_call(
            functools.partial(_bn_mlp_kernel, count=count),
            grid=grid,
            in_specs=[
                pl.BlockSpec((1, BN_BLK, Cin), lambda b_, n_: (b_, n_, 0)),
                pl.BlockSpec((2, Cin), lambda b_, n_: (0, 0)),
                pl.BlockSpec((1, Cin), lambda b_, n_: (0, 0)),
                pl.BlockSpec((1, Cin), lambda b_, n_: (0, 0)),
                pl.BlockSpec((Cout, Cin), lambda b_, n_: (0, 0)),
                pl.BlockSpec((1, Cout), lambda b_, n_: (0, 0)),
            ],
            out_specs=[
                pl.BlockSpec((1, BN_BLK, Cout), lambda b_, n_: (b_, n_, 0)),
                pl.BlockSpec((2, Cout), lambda b_, n_: (0, 0)),
            ],
            out_shape=[
                jax.ShapeDtypeStruct((B, N, Cout), f32),
                jax.ShapeDtypeStruct((2, Cout), f32),
            ],
        )(x, stats, _row2(g), _row2(bt), W, _row2(b))

    z1, s1 = mlp_layer(z0, s0, g0, bt0, W1, b1, H0, H1)
    z2, s2 = mlp_layer(z1, s1, g1, bt1, W2, b2, H1, H2)

    out = pl.pallas_call(
        functools.partial(_bn_relu_kernel, count=count),
        grid=grid,
        in_specs=[
            pl.BlockSpec((1, BN_BLK, H2), lambda b, n: (b, n, 0)),
            pl.BlockSpec((2, H2), lambda b, n: (0, 0)),
            pl.BlockSpec((1, H2), lambda b, n: (0, 0)),
            pl.BlockSpec((1, H2), lambda b, n: (0, 0)),
        ],
        out_specs=pl.BlockSpec((1, BN_BLK, H2), lambda b, n: (b, n, 0)),
        out_shape=jax.ShapeDtypeStruct((B, N, H2), f32),
    )(z2, s2, _row2(g2), _row2(bt2))

    return out
